# Initial kernel scaffold; baseline (speedup 1.0000x reference)
import jax
import jax.numpy as jnp
from jax.experimental import pallas as pl


def kernel(x, token_emb_table):
    raise NotImplementedError("write your pallas kernel here")


# speedup vs baseline: 1.4908x; 1.4908x over previous
import functools

import jax
import jax.numpy as jnp
from jax import lax
from jax.experimental import pallas as pl
from jax.experimental.pallas import tpu as pltpu
from jax.experimental.pallas import tpu_sc as plsc


def _emb_lookup(n_total, vocab, dim):
    info = plsc.get_sparse_core_info()
    nw = info.num_cores * info.num_subcores
    n_per_w = n_total // nw
    chunk = 1600
    n_chunks = n_per_w // chunk
    assert n_per_w % chunk == 0 and n_chunks >= 4

    mesh = plsc.VectorSubcoreMesh(core_axis_name="c", subcore_axis_name="s")

    @functools.partial(
        pl.kernel,
        mesh=mesh,
        compiler_params=pltpu.CompilerParams(use_tc_tiling_on_sc=False),
        out_type=jax.ShapeDtypeStruct((n_total, dim), jnp.float32),
        scratch_types=[
            pltpu.VMEM((2, chunk), jnp.int32),
            pltpu.VMEM((2, chunk, dim), jnp.float32),
            pltpu.SemaphoreType.DMA,
            pltpu.SemaphoreType.DMA,
            pltpu.SemaphoreType.DMA,
            pltpu.SemaphoreType.DMA,
            pltpu.SemaphoreType.DMA,
            pltpu.SemaphoreType.DMA,
        ],
    )
    def emb(idx_hbm, table_hbm, out_hbm, idx_v, rows_v,
            semi0, semi1, semg0, semg1, semo0, semo1):
        semi = (semi0, semi1)
        semg = (semg0, semg1)
        semo = (semo0, semo1)
        wid = lax.axis_index("s") * info.num_cores + lax.axis_index("c")
        base = wid * n_per_w

        idx_cp = [None] * n_chunks
        g_cp = [None] * n_chunks
        o_cp = [None] * n_chunks

        def issue_idx(i):
            off = base + i * chunk
            cp = pltpu.make_async_copy(
                idx_hbm.at[pl.ds(off, chunk)], idx_v.at[i % 2], semi[i % 2])
            cp.start()
            idx_cp[i] = cp

        def issue_gather(i):
            cp = pltpu.make_async_copy(
                table_hbm.at[idx_v.at[i % 2]], rows_v.at[i % 2], semg[i % 2])
            cp.start()
            g_cp[i] = cp

        def issue_out(i):
            off = base + i * chunk
            cp = pltpu.make_async_copy(
                rows_v.at[i % 2], out_hbm.at[pl.ds(off, chunk)], semo[i % 2])
            cp.start()
            o_cp[i] = cp

        issue_idx(0)
        issue_idx(1)
        idx_cp[0].wait()
        issue_gather(0)
        for i in range(n_chunks):
            g_cp[i].wait()
            issue_out(i)
            if i + 2 < n_chunks:
                issue_idx(i + 2)
            if i + 1 < n_chunks:
                idx_cp[i + 1].wait()
                if i >= 1:
                    o_cp[i - 1].wait()
                issue_gather(i + 1)
        o_cp[n_chunks - 2].wait()
        o_cp[n_chunks - 1].wait()

    return emb


def kernel(x, token_emb_table):
    batch, seq = x.shape
    vocab, dim = token_emb_table.shape
    n_total = batch * seq
    idx = x.reshape(n_total).astype(jnp.int32)
    emb = _emb_lookup(n_total, vocab, dim)
    out = emb(idx, token_emb_table)
    return out.reshape(batch, seq, dim)
